# baseline (speedup 1.0000x reference)
import functools

import jax
import jax.numpy as jnp
from jax.experimental import pallas as pl
from jax.experimental.pallas import tpu as pltpu

K_TOP = 64


def _encode_body(x_ref, c_ref, w_ref, b_ref, out_ref):
    u = (x_ref[...] - c_ref[...]).astype(jnp.bfloat16)
    w = w_ref[...].astype(jnp.bfloat16)
    acc = jnp.dot(u, w, preferred_element_type=jnp.float32)
    out_ref[...] = acc + b_ref[...]


def _encode(x, c, w_enc, b_enc, bm=512, bn=2048):
    m, k = x.shape
    n = w_enc.shape[1]
    grid = (n // bn, m // bm)
    return pl.pallas_call(
        _encode_body,
        grid=grid,
        in_specs=[
            pl.BlockSpec((bm, k), lambda j, i: (i, 0)),
            pl.BlockSpec((1, k), lambda j, i: (0, 0)),
            pl.BlockSpec((k, bn), lambda j, i: (0, j)),
            pl.BlockSpec((1, bn), lambda j, i: (0, j)),
        ],
        out_specs=pl.BlockSpec((bm, bn), lambda j, i: (i, j)),
        out_shape=jax.ShapeDtypeStruct((m, n), jnp.float32),
    )(x, c.reshape(1, k), w_enc, b_enc.reshape(1, n))


def _decode_body(z_ref, w_ref, b_ref, out_ref, acc_ref):
    kk = pl.program_id(1)
    zb = z_ref[...].astype(jnp.bfloat16)
    wb = w_ref[...].astype(jnp.bfloat16)
    part = jnp.dot(zb, wb, preferred_element_type=jnp.float32)

    @pl.when(kk == 0)
    def _():
        acc_ref[...] = part

    @pl.when(kk > 0)
    def _():
        acc_ref[...] += part

    @pl.when(kk == pl.num_programs(1) - 1)
    def _():
        out_ref[...] = acc_ref[...] + b_ref[...]


def _decode(z, w_dec, b_last, bm=1024, bk=1024):
    m, n_sae = z.shape
    bm = min(bm, m)
    d = w_dec.shape[1]
    grid = (m // bm, n_sae // bk)
    return pl.pallas_call(
        _decode_body,
        grid=grid,
        in_specs=[
            pl.BlockSpec((bm, bk), lambda i, kk: (i, kk)),
            pl.BlockSpec((bk, d), lambda i, kk: (kk, 0)),
            pl.BlockSpec((1, d), lambda i, kk: (0, 0)),
        ],
        out_specs=pl.BlockSpec((bm, d), lambda i, kk: (i, 0)),
        out_shape=jax.ShapeDtypeStruct((m, d), jnp.float32),
        scratch_shapes=[pltpu.VMEM((bm, d), jnp.float32)],
    )(z, w_dec, b_last.reshape(1, d))


def kernel(x, W_enc, b_enc, W_dec_last, b_dec):
    c = jnp.mean(b_dec, axis=0)
    pre = _encode(x, c, W_enc, b_enc)
    vals, idx = jax.lax.top_k(pre, K_TOP)
    rows = jnp.arange(pre.shape[0])[:, None]
    z = jnp.zeros_like(pre).at[rows, idx].set(vals)
    x_hat = _decode(z, W_dec_last, b_dec[-1])
    return (x_hat, z)


# speedup vs baseline: 4.1594x; 4.1594x over previous
import functools

import jax
import jax.numpy as jnp
from jax import lax
from jax.experimental import pallas as pl
from jax.experimental.pallas import tpu as pltpu
from jax.experimental.pallas import tpu_sc as plsc

K_TOP = 64
_NC = 2
_NS = 16
_L = 16


def _encode_body(x_ref, c_ref, w_ref, b_ref, out_ref):
    u = (x_ref[...] - c_ref[...]).astype(jnp.bfloat16)
    w = w_ref[...].astype(jnp.bfloat16)
    acc = jnp.dot(u, w, preferred_element_type=jnp.float32)
    out_ref[...] = acc + b_ref[...]


def _encode(x, c, w_enc, b_enc, bm=512, bn=2048):
    m, k = x.shape
    n = w_enc.shape[1]
    grid = (n // bn, m // bm)
    return pl.pallas_call(
        _encode_body,
        grid=grid,
        in_specs=[
            pl.BlockSpec((bm, k), lambda j, i: (i, 0)),
            pl.BlockSpec((1, k), lambda j, i: (0, 0)),
            pl.BlockSpec((k, bn), lambda j, i: (0, j)),
            pl.BlockSpec((1, bn), lambda j, i: (0, j)),
        ],
        out_specs=pl.BlockSpec((bm, bn), lambda j, i: (i, j)),
        out_shape=jax.ShapeDtypeStruct((m, n), jnp.float32),
    )(x, c.reshape(1, k), w_enc, b_enc.reshape(1, n))


def _decode_body(z_ref, w_ref, b_ref, out_ref, acc_ref):
    kk = pl.program_id(1)
    zb = z_ref[...].astype(jnp.bfloat16)
    wb = w_ref[...].astype(jnp.bfloat16)
    part = jnp.dot(zb, wb, preferred_element_type=jnp.float32)

    @pl.when(kk == 0)
    def _():
        acc_ref[...] = part

    @pl.when(kk > 0)
    def _():
        acc_ref[...] += part

    @pl.when(kk == pl.num_programs(1) - 1)
    def _():
        out_ref[...] = acc_ref[...] + b_ref[...]


def _decode(z, w_dec, b_last, bm=1024, bk=1024):
    m, n_sae = z.shape
    bm = min(bm, m)
    d = w_dec.shape[1]
    grid = (m // bm, n_sae // bk)
    return pl.pallas_call(
        _decode_body,
        grid=grid,
        in_specs=[
            pl.BlockSpec((bm, bk), lambda i, kk: (i, kk)),
            pl.BlockSpec((bk, d), lambda i, kk: (kk, 0)),
            pl.BlockSpec((1, d), lambda i, kk: (0, 0)),
        ],
        out_specs=pl.BlockSpec((bm, d), lambda i, kk: (i, 0)),
        out_shape=jax.ShapeDtypeStruct((m, d), jnp.float32),
        scratch_shapes=[pltpu.VMEM((bm, d), jnp.float32)],
    )(z, w_dec, b_last.reshape(1, d))


def _sc_topk(pre):
    m, n = pre.shape
    nw = _NC * _NS
    rows_per_w = m // nw
    nv = n // _L
    mesh = plsc.VectorSubcoreMesh(core_axis_name="c", subcore_axis_name="s")

    @functools.partial(
        pl.kernel,
        mesh=mesh,
        out_type=jax.ShapeDtypeStruct((m, n), jnp.float32),
        scratch_types=[
            pltpu.VMEM((n,), jnp.float32),
            pltpu.VMEM((n,), jnp.float32),
            pltpu.VMEM((16 * 256,), jnp.int32),
            pltpu.VMEM((n,), jnp.int32),
            pltpu.VMEM((n,), jnp.int32),
        ],
        compiler_params=pltpu.CompilerParams(needs_layout_passes=False),
    )
    def topk_kernel(pre_hbm, z_hbm, row_v, zrow_v, hist_v, cs_v, ci_v):
        wid = lax.axis_index("s") * _NC + lax.axis_index("c")
        lane = lax.iota(jnp.int32, _L)
        ones = jnp.ones((_L,), jnp.int32)
        zeros_i = jnp.zeros((_L,), jnp.int32)

        def row_body(ri, _):
            row = wid * rows_per_w + ri
            pltpu.sync_copy(pre_hbm.at[row], row_v)

            def zh(j, c):
                hist_v[pl.ds(j * _L, _L)] = zeros_i
                return c

            lax.fori_loop(0, (16 * 256) // _L, zh, 0)

            def p1(i, c):
                v = row_v[pl.ds(i * _L, _L)]
                kb = lax.bitcast_convert_type(v, jnp.int32)
                s = jnp.where(kb < 0, kb ^ 0x7FFFFFFF, kb)
                b = (s >> 24) + 128
                plsc.addupdate_scatter(hist_v, [lane * 256 + b], ones)
                return c

            lax.fori_loop(0, nv, p1, 0)

            def p2(jj, carry):
                acc, b1, ca, found = carry
                j = 15 - jj

                def sl(l, t):
                    return t + hist_v[pl.ds(l * 256 + j * _L, _L)]

                tot = lax.fori_loop(0, 16, sl, zeros_i)
                rt = lax.rev(tot, (0,))
                csum = plsc.cumsum(rt)
                gs = acc + csum
                hit = gs >= K_TOP
                anyhit = jnp.max(gs) >= K_TOP
                lhit = jnp.min(jnp.where(hit, lane, _L))
                bin_cnt = jnp.sum(jnp.where(lane == lhit, rt, 0))
                gs_at = jnp.sum(jnp.where(lane == lhit, gs, 0))
                take = jnp.logical_and(anyhit, jnp.logical_not(found))
                b1 = jnp.where(take, j * _L + 15 - lhit, b1)
                ca = jnp.where(take, gs_at - bin_cnt, ca)
                found = jnp.logical_or(found, anyhit)
                acc = acc + jnp.sum(tot)
                return acc, b1, ca, found

            _, b1, count_above, _ = lax.fori_loop(
                0, 16, p2,
                (jnp.int32(0), jnp.int32(0), jnp.int32(0), False))

            def p3(i, off):
                v = row_v[pl.ds(i * _L, _L)]
                kb = lax.bitcast_convert_type(v, jnp.int32)
                s = jnp.where(kb < 0, kb ^ 0x7FFFFFFF, kb)
                b = (s >> 24) + 128
                zrow_v[pl.ds(i * _L, _L)] = jnp.where(b > b1, v, 0.0)
                cand = b == b1
                ci32 = cand.astype(jnp.int32)
                cnt = plsc.cumsum(ci32)
                pos = off + cnt - 1
                plsc.store_scatter(cs_v, [pos], s & 0xFFFFFF, mask=cand)
                plsc.store_scatter(ci_v, [pos], lane + i * _L, mask=cand)
                return off + jnp.sum(ci32)

            m1 = lax.fori_loop(0, nv, p3, jnp.int32(0))
            nvc = (m1 + (_L - 1)) >> 4
            r = K_TOP - count_above

            def count_ge(t):
                def cb(tt, c):
                    a = cs_v[pl.ds(tt * _L, _L)]
                    lm = (tt * _L + lane) < m1
                    ok = jnp.logical_and(lm, a >= t)
                    return c + jnp.sum(ok.astype(jnp.int32))

                return lax.fori_loop(0, nvc, cb, jnp.int32(0))

            def bis_a(_s, carry):
                lo, hi = carry
                mid = lo + ((hi - lo) >> 1)
                c = count_ge(mid)
                ge = c >= r
                return jnp.where(ge, mid, lo), jnp.where(ge, hi, mid)

            v_lo, _ = lax.fori_loop(
                0, 24, bis_a, (jnp.int32(0), jnp.int32(1 << 24)))
            c_gt = count_ge(v_lo + 1)
            e = r - c_gt

            def tie_le(ii):
                def cb(tt, c):
                    a = cs_v[pl.ds(tt * _L, _L)]
                    ix = ci_v[pl.ds(tt * _L, _L)]
                    lm = (tt * _L + lane) < m1
                    ok = jnp.logical_and(
                        jnp.logical_and(lm, a == v_lo), ix <= ii)
                    return c + jnp.sum(ok.astype(jnp.int32))

                return lax.fori_loop(0, nvc, cb, jnp.int32(0))

            def bis_b(_s, carry):
                lo2, hi2 = carry
                mid = lo2 + ((hi2 - lo2 + 1) >> 1)
                c = tie_le(mid)
                ge = c >= e
                return jnp.where(ge, lo2, mid), jnp.where(ge, mid, hi2)

            _, idx_cut = lax.fori_loop(
                0, 14, bis_b, (jnp.int32(-1), jnp.int32(n - 1)))

            b1s = b1 - 128

            def p5(tt, c):
                a = cs_v[pl.ds(tt * _L, _L)]
                ix = ci_v[pl.ds(tt * _L, _L)]
                lm = (tt * _L + lane) < m1
                keep = jnp.logical_and(
                    lm,
                    jnp.logical_or(
                        a > v_lo,
                        jnp.logical_and(a == v_lo, ix <= idx_cut)))
                s = (b1s << 24) | a
                kb = jnp.where(s < 0, s ^ 0x7FFFFFFF, s)
                val = lax.bitcast_convert_type(kb, jnp.float32)
                plsc.store_scatter(zrow_v, [ix], val, mask=keep)
                return c

            lax.fori_loop(0, nvc, p5, 0)
            pltpu.sync_copy(zrow_v, z_hbm.at[row])
            return _

        lax.fori_loop(0, rows_per_w, row_body, 0)

    return topk_kernel(pre)


def kernel(x, W_enc, b_enc, W_dec_last, b_dec):
    c = jnp.mean(b_dec, axis=0)
    pre = _encode(x, c, W_enc, b_enc)
    z = _sc_topk(pre)
    x_hat = _decode(z, W_dec_last, b_dec[-1])
    return (x_hat, z)


# speedup vs baseline: 4.5587x; 1.0960x over previous
import functools

import jax
import jax.numpy as jnp
from jax import lax
from jax.experimental import pallas as pl
from jax.experimental.pallas import tpu as pltpu
from jax.experimental.pallas import tpu_sc as plsc

K_TOP = 64
_NC = 2
_NS = 16
_L = 16


def _encode_body(x_ref, c_ref, w_ref, b_ref, out_ref):
    u = (x_ref[...] - c_ref[...]).astype(jnp.bfloat16)
    w = w_ref[...].astype(jnp.bfloat16)
    acc = jnp.dot(u, w, preferred_element_type=jnp.float32)
    out_ref[...] = acc + b_ref[...]


def _encode(x, c, w_enc, b_enc, bm=512, bn=2048):
    m, k = x.shape
    n = w_enc.shape[1]
    grid = (n // bn, m // bm)
    return pl.pallas_call(
        _encode_body,
        grid=grid,
        in_specs=[
            pl.BlockSpec((bm, k), lambda j, i: (i, 0)),
            pl.BlockSpec((1, k), lambda j, i: (0, 0)),
            pl.BlockSpec((k, bn), lambda j, i: (0, j)),
            pl.BlockSpec((1, bn), lambda j, i: (0, j)),
        ],
        out_specs=pl.BlockSpec((bm, bn), lambda j, i: (i, j)),
        out_shape=jax.ShapeDtypeStruct((m, n), jnp.float32),
    )(x, c.reshape(1, k), w_enc, b_enc.reshape(1, n))


def _decode_body(z_ref, w_ref, b_ref, out_ref, acc_ref):
    kk = pl.program_id(1)
    zb = z_ref[...].astype(jnp.bfloat16)
    wb = w_ref[...].astype(jnp.bfloat16)
    part = jnp.dot(zb, wb, preferred_element_type=jnp.float32)

    @pl.when(kk == 0)
    def _():
        acc_ref[...] = part

    @pl.when(kk > 0)
    def _():
        acc_ref[...] += part

    @pl.when(kk == pl.num_programs(1) - 1)
    def _():
        out_ref[...] = acc_ref[...] + b_ref[...]


def _decode(z, w_dec, b_last, bm=1024, bk=1024):
    m, n_sae = z.shape
    bm = min(bm, m)
    d = w_dec.shape[1]
    grid = (m // bm, n_sae // bk)
    return pl.pallas_call(
        _decode_body,
        grid=grid,
        in_specs=[
            pl.BlockSpec((bm, bk), lambda i, kk: (i, kk)),
            pl.BlockSpec((bk, d), lambda i, kk: (kk, 0)),
            pl.BlockSpec((1, d), lambda i, kk: (0, 0)),
        ],
        out_specs=pl.BlockSpec((bm, d), lambda i, kk: (i, 0)),
        out_shape=jax.ShapeDtypeStruct((m, d), jnp.float32),
        scratch_shapes=[pltpu.VMEM((bm, d), jnp.float32)],
    )(z, w_dec, b_last.reshape(1, d))


def _sc_topk(pre):
    m, n = pre.shape
    nw = _NC * _NS
    rows_per_w = m // nw
    nv = n // _L
    mesh = plsc.VectorSubcoreMesh(core_axis_name="c", subcore_axis_name="s")

    @functools.partial(
        pl.kernel,
        mesh=mesh,
        out_type=jax.ShapeDtypeStruct((m, n), jnp.float32),
        scratch_types=[
            pltpu.VMEM((n,), jnp.float32),
            pltpu.VMEM((n,), jnp.float32),
            pltpu.VMEM((n,), jnp.float32),
            pltpu.VMEM((n,), jnp.float32),
            pltpu.VMEM((16 * 256,), jnp.int32),
            pltpu.VMEM((n,), jnp.int32),
            pltpu.VMEM((n,), jnp.int32),
            pltpu.SemaphoreType.DMA,
            pltpu.SemaphoreType.DMA,
            pltpu.SemaphoreType.DMA,
            pltpu.SemaphoreType.DMA,
        ],
        compiler_params=pltpu.CompilerParams(needs_layout_passes=False),
    )
    def topk_kernel(pre_hbm, z_hbm, r0, r1, zr0, zr1, hist_v, cs_v, ci_v,
                    si0, si1, so0, so1):
        wid = lax.axis_index("s") * _NC + lax.axis_index("c")
        base = wid * rows_per_w
        lane = lax.iota(jnp.int32, _L)
        laneoff = lane * 256
        ones = jnp.ones((_L,), jnp.int32)
        zeros_i = jnp.zeros((_L,), jnp.int32)

        def process(rv, zv):
            def zh(j, c):
                hist_v[pl.ds(j * _L, _L)] = zeros_i
                return c

            lax.fori_loop(0, (16 * 256) // _L, zh, 0, unroll=8)

            def p1(i, c):
                v = rv[pl.ds(i * _L, _L)]
                kb = lax.bitcast_convert_type(v, jnp.int32)
                s = jnp.where(kb < 0, kb ^ 0x7FFFFFFF, kb)
                b = (s >> 24) + 128
                plsc.addupdate_scatter(hist_v, [laneoff + b], ones)
                return c

            lax.fori_loop(0, nv, p1, 0, unroll=8)

            def p2(jj, carry):
                acc, b1v, cav = carry
                j = 15 - jj

                def sl(l, t):
                    return t + hist_v[pl.ds(l * 256 + j * _L, _L)]

                tot = lax.fori_loop(0, 16, sl, zeros_i, unroll=16)
                rt = lax.rev(tot, (0,))
                csum = plsc.cumsum(rt)
                gs = acc + csum
                hit = gs >= K_TOP
                binidx = j * _L + 15 - lane
                b1v = jnp.maximum(b1v, jnp.where(hit, binidx, -1))
                cav = jnp.minimum(cav, jnp.where(hit, gs - rt, 1 << 30))
                acc = acc + jnp.max(csum)
                return acc, b1v, cav

            _, b1v, cav = lax.fori_loop(
                0, 16, p2,
                (jnp.int32(0), jnp.full((_L,), -1, jnp.int32),
                 jnp.full((_L,), 1 << 30, jnp.int32)))
            b1 = jnp.max(b1v)
            count_above = jnp.min(cav)

            def p3(i, offv):
                v = rv[pl.ds(i * _L, _L)]
                kb = lax.bitcast_convert_type(v, jnp.int32)
                s = jnp.where(kb < 0, kb ^ 0x7FFFFFFF, kb)
                b = (s >> 24) + 128
                zv[pl.ds(i * _L, _L)] = jnp.where(b > b1, v, 0.0)
                cand = b == b1
                cnt = plsc.cumsum(cand.astype(jnp.int32))
                pos = offv + cnt - 1
                plsc.store_scatter(cs_v, [pos], s & 0xFFFFFF, mask=cand)
                plsc.store_scatter(ci_v, [pos], lane + i * _L, mask=cand)
                return offv + plsc.all_reduce_population_count(cand)

            offv = lax.fori_loop(0, nv, p3, zeros_i, unroll=4)
            m1 = jnp.max(offv)
            nvc = (m1 + (_L - 1)) >> 4
            r = K_TOP - count_above

            def count_ge(t):
                def cb(tt, cvec):
                    a = cs_v[pl.ds(tt * _L, _L)]
                    lm = (tt * _L + lane) < m1
                    ok = jnp.logical_and(lm, a >= t)
                    return cvec + ok.astype(jnp.int32)

                return jnp.sum(lax.fori_loop(0, nvc, cb, zeros_i))

            def bis_a(_s, carry):
                lo, hi = carry
                mid = lo + ((hi - lo) >> 1)
                c = count_ge(mid)
                ge = c >= r
                return jnp.where(ge, mid, lo), jnp.where(ge, hi, mid)

            v_lo, _ = lax.fori_loop(
                0, 24, bis_a, (jnp.int32(0), jnp.int32(1 << 24)))
            c_gt = count_ge(v_lo + 1)
            e = r - c_gt

            def tie_le(ii):
                def cb(tt, cvec):
                    a = cs_v[pl.ds(tt * _L, _L)]
                    ix = ci_v[pl.ds(tt * _L, _L)]
                    lm = (tt * _L + lane) < m1
                    ok = jnp.logical_and(
                        jnp.logical_and(lm, a == v_lo), ix <= ii)
                    return cvec + ok.astype(jnp.int32)

                return jnp.sum(lax.fori_loop(0, nvc, cb, zeros_i))

            def bis_b(_s, carry):
                lo2, hi2 = carry
                mid = lo2 + ((hi2 - lo2 + 1) >> 1)
                c = tie_le(mid)
                ge = c >= e
                return jnp.where(ge, lo2, mid), jnp.where(ge, mid, hi2)

            _, idx_cut = lax.fori_loop(
                0, 14, bis_b, (jnp.int32(-1), jnp.int32(n - 1)))

            b1s = b1 - 128

            def p5(tt, c):
                a = cs_v[pl.ds(tt * _L, _L)]
                ix = ci_v[pl.ds(tt * _L, _L)]
                lm = (tt * _L + lane) < m1
                keep = jnp.logical_and(
                    lm,
                    jnp.logical_or(
                        a > v_lo,
                        jnp.logical_and(a == v_lo, ix <= idx_cut)))
                s = (b1s << 24) | a
                kb = jnp.where(s < 0, s ^ 0x7FFFFFFF, s)
                val = lax.bitcast_convert_type(kb, jnp.float32)
                plsc.store_scatter(zv, [ix], val, mask=keep)
                return c

            lax.fori_loop(0, nvc, p5, 0)

        bufs = ((r0, zr0, si0, so0), (r1, zr1, si1, so1))
        pltpu.make_async_copy(pre_hbm.at[base], r0, si0).start()

        def outer(ri2, c):
            for par in range(2):
                rv, zv, si, so = bufs[par]
                orv, _, osi, _ = bufs[1 - par]
                lrow = 2 * ri2 + par
                row = base + lrow
                pltpu.make_async_copy(pre_hbm.at[row], rv, si).wait()

                @pl.when(lrow + 1 < rows_per_w)
                def _():
                    pltpu.make_async_copy(
                        pre_hbm.at[row + 1], orv, osi).start()

                @pl.when(lrow >= 2)
                def _():
                    pltpu.make_async_copy(zv, z_hbm.at[row], so).wait()

                process(rv, zv)
                pltpu.make_async_copy(zv, z_hbm.at[row], so).start()
            return c

        lax.fori_loop(0, rows_per_w // 2, outer, 0)
        pltpu.make_async_copy(zr0, z_hbm.at[base], so0).wait()
        pltpu.make_async_copy(zr1, z_hbm.at[base], so1).wait()

    return topk_kernel(pre)


def kernel(x, W_enc, b_enc, W_dec_last, b_dec):
    c = jnp.mean(b_dec, axis=0)
    pre = _encode(x, c, W_enc, b_enc)
    z = _sc_topk(pre)
    x_hat = _decode(z, W_dec_last, b_dec[-1])
    return (x_hat, z)


# speedup vs baseline: 10.0678x; 2.2085x over previous
import functools

import jax
import jax.numpy as jnp
from jax import lax
from jax.experimental import pallas as pl
from jax.experimental.pallas import tpu as pltpu
from jax.experimental.pallas import tpu_sc as plsc

K_TOP = 64
_NC = 2
_NS = 16
_L = 16


def _encode_body(x_ref, c_ref, w_ref, b_ref, out_ref):
    u = (x_ref[...] - c_ref[...]).astype(jnp.bfloat16)
    w = w_ref[...].astype(jnp.bfloat16)
    acc = jnp.dot(u, w, preferred_element_type=jnp.float32)
    out_ref[...] = acc + b_ref[...]


def _encode(x, c, w_enc, b_enc, bm=512, bn=2048):
    m, k = x.shape
    n = w_enc.shape[1]
    grid = (n // bn, m // bm)
    return pl.pallas_call(
        _encode_body,
        grid=grid,
        in_specs=[
            pl.BlockSpec((bm, k), lambda j, i: (i, 0)),
            pl.BlockSpec((1, k), lambda j, i: (0, 0)),
            pl.BlockSpec((k, bn), lambda j, i: (0, j)),
            pl.BlockSpec((1, bn), lambda j, i: (0, j)),
        ],
        out_specs=pl.BlockSpec((bm, bn), lambda j, i: (i, j)),
        out_shape=jax.ShapeDtypeStruct((m, n), jnp.float32),
    )(x, c.reshape(1, k), w_enc, b_enc.reshape(1, n))


def _decode_body(z_ref, w_ref, b_ref, out_ref, acc_ref):
    kk = pl.program_id(1)
    zb = z_ref[...].astype(jnp.bfloat16)
    wb = w_ref[...].astype(jnp.bfloat16)
    part = jnp.dot(zb, wb, preferred_element_type=jnp.float32)

    @pl.when(kk == 0)
    def _():
        acc_ref[...] = part

    @pl.when(kk > 0)
    def _():
        acc_ref[...] += part

    @pl.when(kk == pl.num_programs(1) - 1)
    def _():
        out_ref[...] = acc_ref[...] + b_ref[...]


def _decode(z, w_dec, b_last, bm=1024, bk=1024):
    m, n_sae = z.shape
    bm = min(bm, m)
    d = w_dec.shape[1]
    grid = (m // bm, n_sae // bk)
    return pl.pallas_call(
        _decode_body,
        grid=grid,
        in_specs=[
            pl.BlockSpec((bm, bk), lambda i, kk: (i, kk)),
            pl.BlockSpec((bk, d), lambda i, kk: (kk, 0)),
            pl.BlockSpec((1, d), lambda i, kk: (0, 0)),
        ],
        out_specs=pl.BlockSpec((bm, d), lambda i, kk: (i, 0)),
        out_shape=jax.ShapeDtypeStruct((m, d), jnp.float32),
        scratch_shapes=[pltpu.VMEM((bm, d), jnp.float32)],
    )(z, w_dec, b_last.reshape(1, d))


def _sc_topk(pre):
    m, n = pre.shape
    nw = _NC * _NS
    rows_per_w = m // nw
    nv = n // _L
    mesh = plsc.VectorSubcoreMesh(core_axis_name="c", subcore_axis_name="s")

    @functools.partial(
        pl.kernel,
        mesh=mesh,
        out_type=jax.ShapeDtypeStruct((m, n), jnp.float32),
        scratch_types=[
            pltpu.VMEM((n,), jnp.float32),
            pltpu.VMEM((n,), jnp.float32),
            pltpu.VMEM((n,), jnp.float32),
            pltpu.VMEM((n,), jnp.float32),
            pltpu.VMEM((16 * 256,), jnp.int32),
            pltpu.VMEM((n,), jnp.int32),
            pltpu.VMEM((n,), jnp.int32),
            pltpu.SemaphoreType.DMA,
            pltpu.SemaphoreType.DMA,
            pltpu.SemaphoreType.DMA,
            pltpu.SemaphoreType.DMA,
        ],
        compiler_params=pltpu.CompilerParams(needs_layout_passes=False),
    )
    def topk_kernel(pre_hbm, z_hbm, r0, r1, zr0, zr1, hist_v, cs_v, ci_v,
                    si0, si1, so0, so1):
        wid = lax.axis_index("s") * _NC + lax.axis_index("c")
        base = wid * rows_per_w
        lane = lax.iota(jnp.int32, _L)
        laneoff = lane * 256
        ones = jnp.ones((_L,), jnp.int32)
        zeros_i = jnp.zeros((_L,), jnp.int32)

        def process(rv, zv):
            @plsc.parallel_loop(0, 16 * 256, _L, unroll=8)
            def _zh(j):
                hist_v[pl.ds(j, _L)] = zeros_i

            @plsc.parallel_loop(0, n, _L, unroll=8)
            def _p1(i):
                v = rv[pl.ds(i, _L)]
                kb = lax.bitcast_convert_type(v, jnp.int32)
                s = jnp.where(kb < 0, kb ^ 0x7FFFFFFF, kb)
                b = (s >> 24) + 128
                plsc.addupdate_scatter(hist_v, [laneoff + b], ones)

            def p2(jj, carry):
                acc, b1v, cav = carry
                j = 15 - jj

                def sl(l, t):
                    return t + hist_v[pl.ds(l * 256 + j * _L, _L)]

                tot = lax.fori_loop(0, 16, sl, zeros_i, unroll=16)
                rt = lax.rev(tot, (0,))
                csum = plsc.cumsum(rt)
                gs = acc + csum
                hit = gs >= K_TOP
                binidx = j * _L + 15 - lane
                b1v = jnp.maximum(b1v, jnp.where(hit, binidx, -1))
                cav = jnp.minimum(cav, jnp.where(hit, gs - rt, 1 << 30))
                acc = acc + jnp.max(csum)
                return acc, b1v, cav

            _, b1v, cav = lax.fori_loop(
                0, 16, p2,
                (jnp.int32(0), jnp.full((_L,), -1, jnp.int32),
                 jnp.full((_L,), 1 << 30, jnp.int32)))
            b1 = jnp.max(b1v)
            count_above = jnp.min(cav)

            @plsc.parallel_loop(0, n, _L, unroll=4, carry=zeros_i)
            def offv(i, off):
                v = rv[pl.ds(i, _L)]
                kb = lax.bitcast_convert_type(v, jnp.int32)
                s = jnp.where(kb < 0, kb ^ 0x7FFFFFFF, kb)
                b = (s >> 24) + 128
                zv[pl.ds(i, _L)] = jnp.where(b > b1, v, 0.0)
                cand = b == b1
                cnt = plsc.cumsum(cand.astype(jnp.int32))
                pos = off + cnt - 1
                plsc.store_scatter(cs_v, [pos], s & 0xFFFFFF, mask=cand)
                plsc.store_scatter(ci_v, [pos], lane + i, mask=cand)
                return off + plsc.all_reduce_population_count(cand)

            m1 = jnp.max(offv)
            nvc = (m1 + (_L - 1)) >> 4
            r = K_TOP - count_above

            def count_ge(t):
                def cb(tt, cvec):
                    a = cs_v[pl.ds(tt * _L, _L)]
                    lm = (tt * _L + lane) < m1
                    ok = jnp.logical_and(lm, a >= t)
                    return cvec + ok.astype(jnp.int32)

                return jnp.sum(lax.fori_loop(0, nvc, cb, zeros_i))

            def bis_a(_s, carry):
                lo, hi = carry
                mid = lo + ((hi - lo) >> 1)
                c = count_ge(mid)
                ge = c >= r
                return jnp.where(ge, mid, lo), jnp.where(ge, hi, mid)

            v_lo, _ = lax.fori_loop(
                0, 24, bis_a, (jnp.int32(0), jnp.int32(1 << 24)))
            c_gt = count_ge(v_lo + 1)
            e = r - c_gt

            def tie_le(ii):
                def cb(tt, cvec):
                    a = cs_v[pl.ds(tt * _L, _L)]
                    ix = ci_v[pl.ds(tt * _L, _L)]
                    lm = (tt * _L + lane) < m1
                    ok = jnp.logical_and(
                        jnp.logical_and(lm, a == v_lo), ix <= ii)
                    return cvec + ok.astype(jnp.int32)

                return jnp.sum(lax.fori_loop(0, nvc, cb, zeros_i))

            def bis_b(_s, carry):
                lo2, hi2 = carry
                mid = lo2 + ((hi2 - lo2 + 1) >> 1)
                c = tie_le(mid)
                ge = c >= e
                return jnp.where(ge, lo2, mid), jnp.where(ge, mid, hi2)

            _, idx_cut = lax.fori_loop(
                0, 14, bis_b, (jnp.int32(-1), jnp.int32(n - 1)))

            b1s = b1 - 128

            def p5(tt, c):
                a = cs_v[pl.ds(tt * _L, _L)]
                ix = ci_v[pl.ds(tt * _L, _L)]
                lm = (tt * _L + lane) < m1
                keep = jnp.logical_and(
                    lm,
                    jnp.logical_or(
                        a > v_lo,
                        jnp.logical_and(a == v_lo, ix <= idx_cut)))
                s = (b1s << 24) | a
                kb = jnp.where(s < 0, s ^ 0x7FFFFFFF, s)
                val = lax.bitcast_convert_type(kb, jnp.float32)
                plsc.store_scatter(zv, [ix], val, mask=keep)
                return c

            lax.fori_loop(0, nvc, p5, 0)

        bufs = ((r0, zr0, si0, so0), (r1, zr1, si1, so1))
        pltpu.make_async_copy(pre_hbm.at[base], r0, si0).start()

        def outer(ri2, c):
            for par in range(2):
                rv, zv, si, so = bufs[par]
                orv, _, osi, _ = bufs[1 - par]
                lrow = 2 * ri2 + par
                row = base + lrow
                pltpu.make_async_copy(pre_hbm.at[row], rv, si).wait()

                @pl.when(lrow + 1 < rows_per_w)
                def _():
                    pltpu.make_async_copy(
                        pre_hbm.at[row + 1], orv, osi).start()

                @pl.when(lrow >= 2)
                def _():
                    pltpu.make_async_copy(zv, z_hbm.at[row], so).wait()

                process(rv, zv)
                pltpu.make_async_copy(zv, z_hbm.at[row], so).start()
            return c

        lax.fori_loop(0, rows_per_w // 2, outer, 0)
        pltpu.make_async_copy(zr0, z_hbm.at[base], so0).wait()
        pltpu.make_async_copy(zr1, z_hbm.at[base], so1).wait()

    return topk_kernel(pre)


def kernel(x, W_enc, b_enc, W_dec_last, b_dec):
    c = jnp.mean(b_dec, axis=0)
    pre = _encode(x, c, W_enc, b_enc)
    z = _sc_topk(pre)
    x_hat = _decode(z, W_dec_last, b_dec[-1])
    return (x_hat, z)
